# baseline (speedup 1.0000x reference)
import functools

import jax
import jax.numpy as jnp
from jax import lax
from jax.experimental import pallas as pl
from jax.experimental.pallas import tpu as pltpu
from jax.experimental.pallas import tpu_sc as plsc

_BLK = 512


def _vq_argmin_kernel(z_ref, c_ref, k_ref, csq_ref, c2_ref):
    @pl.when(pl.program_id(0) == 0)
    def _():
        c0 = c_ref[...]
        csq_ref[...] = jnp.sum(c0 * c0, axis=1, keepdims=True)
        c2_ref[...] = -2.0 * c0

    K = c_ref.shape[0]
    z = z_ref[...]
    z_sq = jnp.sum(z * z, axis=1, keepdims=True)
    z_sq_t = jnp.transpose(z_sq)
    dots2 = jax.lax.dot_general(
        c2_ref[...], z, (((1,), (1,)), ((), ())),
        preferred_element_type=jnp.float32)
    score = (z_sq_t + dots2) + csq_ref[...]
    m2 = jnp.min(score, axis=0, keepdims=True)
    d = jnp.sqrt(jnp.maximum(m2, 0.0))
    db = jax.lax.bitcast_convert_type(d, jnp.int32)
    u = jax.lax.bitcast_convert_type(db + 1, jnp.float32) - d
    cc = d * 4097.0
    dh = cc - (cc - d)
    dl = d - dh
    r = d * d
    e = ((dh * dh - r) + 2.0 * (dh * dl)) + dl * dl
    s = e + d * u
    merge = (score - r) <= s
    iota0 = jax.lax.broadcasted_iota(jnp.int32, score.shape, 0)
    cand = jnp.where(merge, iota0, K)
    k_ref[0, 0, :] = jnp.min(cand, axis=0).astype(jnp.int32)


def _make_sc_gather(N, nw):
    bpw = N // nw
    nchunk = bpw // 128
    mesh = plsc.VectorSubcoreMesh(core_axis_name="c", subcore_axis_name="s")

    @functools.partial(
        pl.kernel,
        out_type=jax.ShapeDtypeStruct((N, 128), jnp.float32),
        mesh=mesh,
        scratch_types=[
            pltpu.VMEM((nchunk, 128), jnp.int32),
            pltpu.VMEM((bpw, 128), jnp.float32),
            pltpu.SemaphoreType.DMA,
        ],
    )
    def _gather(table_hbm, idx_hbm, out_hbm, idx_v, rows_v, sem):
        nc = plsc.get_sparse_core_info().num_cores
        wid = lax.axis_index("s") * nc + lax.axis_index("c")
        base = wid * bpw
        pltpu.sync_copy(idx_hbm.at[pl.ds(wid * nchunk, nchunk)], idx_v)
        copies = [
            pltpu.async_copy(
                table_hbm.at[idx_v.at[j]],
                rows_v.at[pl.ds(j * 128, 128)], sem)
            for j in range(nchunk)
        ]
        for cp in copies:
            cp.wait()
        pltpu.sync_copy(rows_v, out_hbm.at[pl.ds(base, bpw)])

    return _gather


@jax.jit
def kernel(inputs, lookup_table):
    B, H, W, D = inputs.shape
    N = B * H * W
    K = lookup_table.shape[0]
    z = inputs.reshape(N, D)
    nblk = N // _BLK
    k3 = pl.pallas_call(
        _vq_argmin_kernel,
        grid=(nblk,),
        in_specs=[
            pl.BlockSpec((_BLK, D), lambda i: (i, 0)),
            pl.BlockSpec((K, D), lambda i: (0, 0)),
        ],
        out_specs=pl.BlockSpec((1, 1, _BLK), lambda i: (i, 0, 0)),
        out_shape=jax.ShapeDtypeStruct((nblk, 1, _BLK), jnp.int32),
        scratch_shapes=[pltpu.VMEM((K, 1), jnp.float32),
                        pltpu.VMEM((K, D), jnp.float32)],
    )(z, lookup_table)
    idx2 = k3.reshape(N // 128, 128)

    info = plsc.get_sparse_core_info()
    nw = info.num_cores * info.num_subcores
    table_pad = jnp.pad(lookup_table, ((0, 0), (0, 128 - D)))
    zq128 = _make_sc_gather(N, nw)(table_pad, idx2)
    zq = zq128[:, :D]

    return (k3.reshape(B, H, W), zq.reshape(B, H, W, D))


# speedup vs baseline: 1.1089x; 1.1089x over previous
import functools

import jax
import jax.numpy as jnp
from jax import lax
from jax.experimental import pallas as pl
from jax.experimental.pallas import tpu as pltpu
from jax.experimental.pallas import tpu_sc as plsc

_BLK = 512


def _vq_argmin_kernel(z_ref, c_ref, k_ref, csq_ref, c2_ref):
    @pl.when(pl.program_id(0) == 0)
    def _():
        c0 = c_ref[...]
        csq_ref[...] = jnp.sum(c0 * c0, axis=1, keepdims=True)
        c2_ref[...] = -2.0 * c0

    K = c_ref.shape[0]
    z = z_ref[...]
    z_sq = jnp.sum(z * z, axis=1, keepdims=True)
    z_sq_t = jnp.transpose(z_sq)
    dots2 = jax.lax.dot_general(
        c2_ref[...], z, (((1,), (1,)), ((), ())),
        preferred_element_type=jnp.float32)
    score = (z_sq_t + dots2) + csq_ref[...]
    dist = jnp.sqrt(jnp.maximum(score, 0.0))
    k_ref[0, 0, :] = jnp.argmin(dist, axis=0).astype(jnp.int32)


def _make_sc_gather(N, nw):
    bpw = N // nw
    nchunk = bpw // 128
    mesh = plsc.VectorSubcoreMesh(core_axis_name="c", subcore_axis_name="s")

    @functools.partial(
        pl.kernel,
        out_type=jax.ShapeDtypeStruct((N, 128), jnp.float32),
        mesh=mesh,
        scratch_types=[
            pltpu.VMEM((nchunk, 128), jnp.int32),
            pltpu.VMEM((bpw, 128), jnp.float32),
            pltpu.SemaphoreType.DMA,
        ],
    )
    def _gather(table_hbm, idx_hbm, out_hbm, idx_v, rows_v, sem):
        nc = plsc.get_sparse_core_info().num_cores
        wid = lax.axis_index("s") * nc + lax.axis_index("c")
        base = wid * bpw
        pltpu.sync_copy(idx_hbm.at[pl.ds(wid * nchunk, nchunk)], idx_v)
        copies = [
            pltpu.async_copy(
                table_hbm.at[idx_v.at[j]],
                rows_v.at[pl.ds(j * 128, 128)], sem)
            for j in range(nchunk)
        ]
        for cp in copies:
            cp.wait()
        pltpu.sync_copy(rows_v, out_hbm.at[pl.ds(base, bpw)])

    return _gather


@jax.jit
def kernel(inputs, lookup_table):
    B, H, W, D = inputs.shape
    N = B * H * W
    K = lookup_table.shape[0]
    z = inputs.reshape(N, D)
    nblk = N // _BLK
    k3 = pl.pallas_call(
        _vq_argmin_kernel,
        grid=(nblk,),
        in_specs=[
            pl.BlockSpec((_BLK, D), lambda i: (i, 0)),
            pl.BlockSpec((K, D), lambda i: (0, 0)),
        ],
        out_specs=pl.BlockSpec((1, 1, _BLK), lambda i: (i, 0, 0)),
        out_shape=jax.ShapeDtypeStruct((nblk, 1, _BLK), jnp.int32),
        scratch_shapes=[pltpu.VMEM((K, 1), jnp.float32),
                        pltpu.VMEM((K, D), jnp.float32)],
    )(z, lookup_table)
    idx2 = k3.reshape(N // 128, 128)

    info = plsc.get_sparse_core_info()
    nw = info.num_cores * info.num_subcores
    table_pad = jnp.pad(lookup_table, ((0, 0), (0, 128 - D)))
    zq128 = _make_sc_gather(N, nw)(table_pad, idx2)
    zq = zq128[:, :D]

    return (k3.reshape(B, H, W), zq.reshape(B, H, W, D))
